# Initial kernel scaffold; baseline (speedup 1.0000x reference)
import jax
import jax.numpy as jnp
from jax.experimental import pallas as pl


def kernel(need_replace, union_feature, all_items, sample_items, W, b):
    raise NotImplementedError("write your pallas kernel here")


# speedup vs baseline: 2.3759x; 2.3759x over previous
import functools

import jax
import jax.numpy as jnp
from jax import lax
from jax.experimental import pallas as pl
from jax.experimental.pallas import tpu as pltpu
from jax.experimental.pallas import tpu_sc as plsc

_B = 4096
_S = 200
_SP = 256
_D = 128
_NC = 2
_NS = 16
_NW = _NC * _NS
_RPW = _B // _NW
_R = 16


def _sc_gather(user_ids, item_ids, all_items, sample_items_padded):
    mesh = plsc.VectorSubcoreMesh(core_axis_name="c", subcore_axis_name="s")

    @functools.partial(
        pl.kernel,
        out_type=(
            jax.ShapeDtypeStruct((_B, _S, _D), jnp.float32),
            jax.ShapeDtypeStruct((_B, _SP), jnp.int32),
            jax.ShapeDtypeStruct((_B, _D), jnp.float32),
        ),
        mesh=mesh,
        scratch_types=[
            pltpu.VMEM((_RPW,), jnp.int32),
            pltpu.VMEM((_RPW,), jnp.int32),
            pltpu.VMEM((_RPW, _SP), jnp.int32),
            pltpu.VMEM((_RPW, _D), jnp.float32),
            pltpu.VMEM((128,), jnp.int32),
            pltpu.VMEM((72,), jnp.int32),
            pltpu.VMEM((_S, _D), jnp.float32),
            pltpu.SemaphoreType.DMA,
            pltpu.SemaphoreType.DMA,
        ],
    )
    def k(uid_hbm, iid_hbm, items_hbm, samples_hbm, f_out, samp_out, emb_out,
          uid_v, iid_v, samp_v, emb_v, idx_a, idx_b, rows_v, sem0, sem1):
        wid = lax.axis_index("s") * _NC + lax.axis_index("c")
        base = wid * _RPW
        pltpu.sync_copy(uid_hbm.at[pl.ds(base, _RPW)], uid_v)
        pltpu.sync_copy(iid_hbm.at[pl.ds(base, _RPW)], iid_v)
        pltpu.async_copy(samples_hbm.at[uid_v], samp_v, sem0).wait()
        pltpu.sync_copy(samp_v, samp_out.at[pl.ds(base, _RPW)])
        pltpu.async_copy(items_hbm.at[iid_v], emb_v, sem0).wait()
        pltpu.sync_copy(emb_v, emb_out.at[pl.ds(base, _RPW)])

        def row_body(r, carry):
            cp1 = pltpu.async_copy(
                items_hbm.at[samp_v.at[r, pl.ds(0, 128)]],
                rows_v.at[pl.ds(0, 128)], sem0)
            cp2 = pltpu.async_copy(
                items_hbm.at[samp_v.at[r, pl.ds(128, 72)]],
                rows_v.at[pl.ds(128, 72)], sem1)
            cp1.wait()
            cp2.wait()
            pltpu.sync_copy(rows_v, f_out.at[base + r])
            return carry

        lax.fori_loop(0, _RPW, row_body, 0)

    return k(user_ids, item_ids, all_items, sample_items_padded)


_SC_CHUNK = 40


def _tc_body(f_ref, emb_ref, uif_ref, samp_ref, g_ref, out1_ref, out2_ref, acc_ref):
    embb = emb_ref[...]
    uifb = uif_ref[...]
    sampb = samp_ref[...][:, :_S]
    gb = g_ref[...]

    rs_parts, ps_parts = [], []
    for s0 in range(0, _S, _SC_CHUNK):
        Fc = f_ref[:, s0:s0 + _SC_CHUNK, :]
        rs_parts.append(jnp.sum(Fc * embb[:, None, :], axis=-1))
        ps_parts.append(jnp.sum(Fc * uifb[:, None, :], axis=-1))
    rs = jnp.concatenate(rs_parts, axis=1)
    ps = jnp.concatenate(ps_parts, axis=1)

    logits = (ps + gb) / jnp.float32(1e-4)
    m = jnp.max(logits, axis=-1, keepdims=True)
    e = jnp.exp(logits - m)
    ysoft = e / jnp.sum(e, axis=-1, keepdims=True)
    ym = jnp.max(ysoft, axis=-1, keepdims=True)
    sidx = lax.broadcasted_iota(jnp.int32, (_R, _S), 1)
    amax = jnp.min(jnp.where(ysoft == ym, sidx, _S), axis=-1, keepdims=True)
    yhard = (sidx == amax).astype(jnp.float32)
    prob = yhard - ysoft + ysoft

    p_sel = jnp.sum(prob, axis=-1)

    rs_am = jnp.sum(jnp.where(sidx == amax, rs, 0.0), axis=-1, keepdims=True)
    rank_am = jnp.sum(
        ((rs < rs_am) | ((rs == rs_am) & (sidx < amax))).astype(jnp.float32),
        axis=-1)

    bits = lax.bitcast_convert_type(rs, jnp.uint32)
    keys = bits ^ jnp.where(bits >= jnp.uint32(0x80000000),
                            jnp.uint32(0xFFFFFFFF), jnp.uint32(0x80000000))
    lo = jnp.zeros((_R, 1), jnp.uint32)
    for bit in range(31, -1, -1):
        cand = lo | jnp.uint32(1 << bit)
        cl = jnp.sum((keys < cand).astype(jnp.int32), axis=-1, keepdims=True)
        lo = jnp.where(cl <= amax, cand, lo)
    kstar = lo
    cls = jnp.sum((keys < kstar).astype(jnp.int32), axis=-1, keepdims=True)
    occ = amax - cls
    match = (keys == kstar)
    c = match.astype(jnp.int32)
    for sh in (1, 2, 4, 8, 16, 32, 64, 128):
        c = c + jnp.pad(c, ((0, 0), (sh, 0)))[:, :_S]
    jmask = match & (c == occ + 1)
    chosen = jnp.sum(jnp.where(jmask, sampb.astype(jnp.float32), 0.0), axis=-1)
    out1_ref[...] = (chosen * p_sel).astype(jnp.int32)[:, None]

    o2 = jnp.zeros((_R, _D), jnp.float32)
    for s0 in range(0, _S, _SC_CHUNK):
        Fc = f_ref[:, s0:s0 + _SC_CHUNK, :]
        o2 = o2 + jnp.sum(Fc * prob[:, s0:s0 + _SC_CHUNK, None], axis=1)
    out2_ref[...] = o2

    pos = rank_am * p_sel + 1.0
    sim = pos / jnp.float32(200.0)

    @pl.when(pl.program_id(0) == 0)
    def _init():
        acc_ref[...] = jnp.zeros_like(acc_ref)

    sa = jnp.sum(jnp.abs(sim - jnp.float32(0.5)))
    ss = jnp.sum(sim)
    r8 = lax.broadcasted_iota(jnp.int32, (8, 2), 0)
    c2 = lax.broadcasted_iota(jnp.int32, (8, 2), 1)
    upd = (jnp.where((r8 == 0) & (c2 == 0), sa, 0.0)
           + jnp.where((r8 == 0) & (c2 == 1), ss, 0.0))
    acc_ref[...] += upd


def _tc_compute(F, emb, uif, samp_pad, g):
    grid = (_B // _R,)
    return pl.pallas_call(
        _tc_body,
        grid=grid,
        in_specs=[
            pl.BlockSpec((_R, _S, _D), lambda i: (i, 0, 0)),
            pl.BlockSpec((_R, _D), lambda i: (i, 0)),
            pl.BlockSpec((_R, _D), lambda i: (i, 0)),
            pl.BlockSpec((_R, _SP), lambda i: (i, 0)),
            pl.BlockSpec((_R, _S), lambda i: (i, 0)),
        ],
        out_specs=[
            pl.BlockSpec((_R, 1), lambda i: (i, 0)),
            pl.BlockSpec((_R, _D), lambda i: (i, 0)),
            pl.BlockSpec((8, 2), lambda i: (0, 0)),
        ],
        out_shape=[
            jax.ShapeDtypeStruct((_B, 1), jnp.int32),
            jax.ShapeDtypeStruct((_B, _D), jnp.float32),
            jax.ShapeDtypeStruct((8, 2), jnp.float32),
        ],
        compiler_params=pltpu.CompilerParams(
            dimension_semantics=("arbitrary",),
        ),
    )(F, emb, uif, samp_pad, g)


def kernel(need_replace, union_feature, all_items, sample_items, W, b):
    user_ids = need_replace[:, 0]
    item_ids = need_replace[:, 1]
    uif = (union_feature @ W.T + b)
    u = jax.random.uniform(jax.random.key(42), (_B, _S), minval=1e-9, maxval=1.0)
    g = -jnp.log(-jnp.log(u))
    samp_src = jnp.pad(sample_items, ((0, 0), (0, _SP - _S)))
    F, samp_pad, emb = _sc_gather(user_ids, item_ids, all_items, samp_src)
    out1, out2, acc = _tc_compute(F, emb, uif, samp_pad, g)
    loss = acc[0, 0] / jnp.float32(_B)
    mean_sim = acc[0, 1] / jnp.float32(_B)
    return (out1.reshape(_B), out2, loss, mean_sim)


# speedup vs baseline: 3.3015x; 1.3896x over previous
import functools

import jax
import jax.numpy as jnp
from jax import lax
from jax.experimental import pallas as pl
from jax.experimental.pallas import tpu as pltpu
from jax.experimental.pallas import tpu_sc as plsc

_B = 4096
_S = 200
_SP = 256
_D = 128
_NC = 2
_NS = 16
_NW = _NC * _NS
_RPW = _B // _NW
_R = 16


def _sc_gather(user_ids, item_ids, all_items, sample_items_padded):
    mesh = plsc.VectorSubcoreMesh(core_axis_name="c", subcore_axis_name="s")

    @functools.partial(
        pl.kernel,
        out_type=(
            jax.ShapeDtypeStruct((_B, _S, _D), jnp.float32),
            jax.ShapeDtypeStruct((_B, _SP), jnp.int32),
            jax.ShapeDtypeStruct((_B, _D), jnp.float32),
        ),
        mesh=mesh,
        scratch_types=[
            pltpu.VMEM((_RPW,), jnp.int32),
            pltpu.VMEM((_RPW,), jnp.int32),
            pltpu.VMEM((_RPW, _SP), jnp.int32),
            pltpu.VMEM((_RPW, _D), jnp.float32),
            pltpu.VMEM((128,), jnp.int32),
            pltpu.VMEM((72,), jnp.int32),
            pltpu.VMEM((_S, _D), jnp.float32),
            pltpu.SemaphoreType.DMA,
            pltpu.SemaphoreType.DMA,
        ],
    )
    def k(uid_hbm, iid_hbm, items_hbm, samples_hbm, f_out, samp_out, emb_out,
          uid_v, iid_v, samp_v, emb_v, idx_a, idx_b, rows_v, sem0, sem1):
        wid = lax.axis_index("s") * _NC + lax.axis_index("c")
        base = wid * _RPW
        pltpu.sync_copy(uid_hbm.at[pl.ds(base, _RPW)], uid_v)
        pltpu.sync_copy(iid_hbm.at[pl.ds(base, _RPW)], iid_v)
        pltpu.async_copy(samples_hbm.at[uid_v], samp_v, sem0).wait()
        pltpu.sync_copy(samp_v, samp_out.at[pl.ds(base, _RPW)])
        pltpu.async_copy(items_hbm.at[iid_v], emb_v, sem0).wait()
        pltpu.sync_copy(emb_v, emb_out.at[pl.ds(base, _RPW)])

        def row_body(r, carry):
            cp1 = pltpu.async_copy(
                items_hbm.at[samp_v.at[r, pl.ds(0, 128)]],
                rows_v.at[pl.ds(0, 128)], sem0)
            cp2 = pltpu.async_copy(
                items_hbm.at[samp_v.at[r, pl.ds(128, 72)]],
                rows_v.at[pl.ds(128, 72)], sem1)
            cp1.wait()
            cp2.wait()
            pltpu.sync_copy(rows_v, f_out.at[base + r])
            return carry

        lax.fori_loop(0, _RPW, row_body, 0)

    return k(user_ids, item_ids, all_items, sample_items_padded)


_SC_CHUNK = 40
_JC = 8


def _tc_body(f_ref, emb_ref, uif_ref, samp_ref, g_ref, out1_ref, out2_ref, acc_ref):
    embb = emb_ref[...]
    uifb = uif_ref[...]
    sampb = samp_ref[...][:, :_S]
    gb = g_ref[...]

    rs_parts, ps_parts = [], []
    for s0 in range(0, _S, _SC_CHUNK):
        Fc = f_ref[:, s0:s0 + _SC_CHUNK, :]
        rs_parts.append(jnp.sum(Fc * embb[:, None, :], axis=-1))
        ps_parts.append(jnp.sum(Fc * uifb[:, None, :], axis=-1))
    rs = jnp.concatenate(rs_parts, axis=1)
    ps = jnp.concatenate(ps_parts, axis=1)

    logits = (ps + gb) / jnp.float32(1e-4)
    m = jnp.max(logits, axis=-1, keepdims=True)
    e = jnp.exp(logits - m)
    ysoft = e / jnp.sum(e, axis=-1, keepdims=True)
    ym = jnp.max(ysoft, axis=-1, keepdims=True)
    sidx = lax.broadcasted_iota(jnp.int32, (_R, _S), 1)
    amax = jnp.min(jnp.where(ysoft == ym, sidx, _S), axis=-1, keepdims=True)
    yhard = (sidx == amax).astype(jnp.float32)
    prob = yhard - ysoft + ysoft

    p_sel = jnp.sum(prob, axis=-1)

    bits = lax.bitcast_convert_type(rs, jnp.uint32)
    keys = bits ^ jnp.where(bits >= jnp.uint32(0x80000000),
                            jnp.uint32(0xFFFFFFFF), jnp.uint32(0x80000000))
    ones_s = jnp.ones((_S, 1), jnp.float32)
    kidx3 = lax.broadcasted_iota(jnp.int32, (1, 1, _S), 2)
    kk = keys[:, None, :]
    cnt_parts = []
    for j0 in range(0, _S, _JC):
        kj = keys[:, j0:j0 + _JC][:, :, None]
        jj = j0 + lax.broadcasted_iota(jnp.int32, (1, _JC, 1), 1)
        ind = ((kk < kj) | ((kk == kj) & (kidx3 < jj))).astype(jnp.float32)
        cc = lax.dot_general(ind.reshape(_R * _JC, _S), ones_s,
                             (((1,), (0,)), ((), ())))
        cnt_parts.append(cc.reshape(_R, _JC))
    cnt = jnp.concatenate(cnt_parts, axis=1)

    amax_f = amax.astype(jnp.float32)
    rank_am = jnp.sum(jnp.where(sidx == amax, cnt, 0.0), axis=-1)
    chosen = jnp.sum(jnp.where(cnt == amax_f, sampb.astype(jnp.float32), 0.0),
                     axis=-1)
    out1_ref[...] = (chosen * p_sel).astype(jnp.int32)[:, None]

    Fb = f_ref[...]
    out2_ref[...] = lax.dot_general(
        prob, Fb, (((1,), (1,)), ((0,), (0,))),
        precision=lax.Precision.HIGHEST)

    pos = rank_am * p_sel + 1.0
    sim = pos / jnp.float32(200.0)

    @pl.when(pl.program_id(0) == 0)
    def _init():
        acc_ref[...] = jnp.zeros_like(acc_ref)

    sa = jnp.sum(jnp.abs(sim - jnp.float32(0.5)))
    ss = jnp.sum(sim)
    r8 = lax.broadcasted_iota(jnp.int32, (8, 2), 0)
    c2 = lax.broadcasted_iota(jnp.int32, (8, 2), 1)
    upd = (jnp.where((r8 == 0) & (c2 == 0), sa, 0.0)
           + jnp.where((r8 == 0) & (c2 == 1), ss, 0.0))
    acc_ref[...] += upd


def _tc_compute(F, emb, uif, samp_pad, g):
    grid = (_B // _R,)
    return pl.pallas_call(
        _tc_body,
        grid=grid,
        in_specs=[
            pl.BlockSpec((_R, _S, _D), lambda i: (i, 0, 0)),
            pl.BlockSpec((_R, _D), lambda i: (i, 0)),
            pl.BlockSpec((_R, _D), lambda i: (i, 0)),
            pl.BlockSpec((_R, _SP), lambda i: (i, 0)),
            pl.BlockSpec((_R, _S), lambda i: (i, 0)),
        ],
        out_specs=[
            pl.BlockSpec((_R, 1), lambda i: (i, 0)),
            pl.BlockSpec((_R, _D), lambda i: (i, 0)),
            pl.BlockSpec((8, 2), lambda i: (0, 0)),
        ],
        out_shape=[
            jax.ShapeDtypeStruct((_B, 1), jnp.int32),
            jax.ShapeDtypeStruct((_B, _D), jnp.float32),
            jax.ShapeDtypeStruct((8, 2), jnp.float32),
        ],
        compiler_params=pltpu.CompilerParams(
            dimension_semantics=("arbitrary",),
        ),
    )(F, emb, uif, samp_pad, g)


def kernel(need_replace, union_feature, all_items, sample_items, W, b):
    user_ids = need_replace[:, 0]
    item_ids = need_replace[:, 1]
    uif = (union_feature @ W.T + b)
    u = jax.random.uniform(jax.random.key(42), (_B, _S), minval=1e-9, maxval=1.0)
    g = -jnp.log(-jnp.log(u))
    samp_src = jnp.pad(sample_items, ((0, 0), (0, _SP - _S)))
    F, samp_pad, emb = _sc_gather(user_ids, item_ids, all_items, samp_src)
    out1, out2, acc = _tc_compute(F, emb, uif, samp_pad, g)
    loss = acc[0, 0] / jnp.float32(_B)
    mean_sim = acc[0, 1] / jnp.float32(_B)
    return (out1.reshape(_B), out2, loss, mean_sim)


# speedup vs baseline: 3.4431x; 1.0429x over previous
import functools

import jax
import jax.numpy as jnp
from jax import lax
from jax.experimental import pallas as pl
from jax.experimental.pallas import tpu as pltpu
from jax.experimental.pallas import tpu_sc as plsc

_B = 4096
_S = 200
_SP = 256
_D = 128
_NC = 2
_NS = 16
_NW = _NC * _NS
_RPW = _B // _NW
_R = 16


def _sc_gather(user_ids, item_ids, all_items, sample_items_padded):
    mesh = plsc.VectorSubcoreMesh(core_axis_name="c", subcore_axis_name="s")

    @functools.partial(
        pl.kernel,
        out_type=(
            jax.ShapeDtypeStruct((_B, _S, _D), jnp.float32),
            jax.ShapeDtypeStruct((_B, _SP), jnp.int32),
            jax.ShapeDtypeStruct((_B, _D), jnp.float32),
        ),
        mesh=mesh,
        scratch_types=[
            pltpu.VMEM((_RPW,), jnp.int32),
            pltpu.VMEM((_RPW,), jnp.int32),
            pltpu.VMEM((_RPW, _SP), jnp.int32),
            pltpu.VMEM((_RPW, _D), jnp.float32),
            pltpu.VMEM((_S, _D), jnp.float32),
            pltpu.VMEM((_S, _D), jnp.float32),
            pltpu.SemaphoreType.DMA,
            pltpu.SemaphoreType.DMA,
            pltpu.SemaphoreType.DMA,
            pltpu.SemaphoreType.DMA,
        ],
    )
    def k(uid_hbm, iid_hbm, items_hbm, samples_hbm, f_out, samp_out, emb_out,
          uid_v, iid_v, samp_v, emb_v, rows_a, rows_b, semA1, semA2, semB1, semB2):
        wid = lax.axis_index("s") * _NC + lax.axis_index("c")
        base = wid * _RPW
        pltpu.sync_copy(uid_hbm.at[pl.ds(base, _RPW)], uid_v)
        pltpu.sync_copy(iid_hbm.at[pl.ds(base, _RPW)], iid_v)
        pltpu.async_copy(samples_hbm.at[uid_v], samp_v, semA1).wait()
        pltpu.sync_copy(samp_v, samp_out.at[pl.ds(base, _RPW)])
        pltpu.async_copy(items_hbm.at[iid_v], emb_v, semA1).wait()
        pltpu.sync_copy(emb_v, emb_out.at[pl.ds(base, _RPW)])

        def fire(r, rv, s1, s2):
            pltpu.async_copy(items_hbm.at[samp_v.at[r, pl.ds(0, 128)]],
                             rv.at[pl.ds(0, 128)], s1)
            pltpu.async_copy(items_hbm.at[samp_v.at[r, pl.ds(128, 72)]],
                             rv.at[pl.ds(128, 72)], s2)

        def wait(r, rv, s1, s2):
            pltpu.make_async_copy(items_hbm.at[samp_v.at[r, pl.ds(0, 128)]],
                                  rv.at[pl.ds(0, 128)], s1).wait()
            pltpu.make_async_copy(items_hbm.at[samp_v.at[r, pl.ds(128, 72)]],
                                  rv.at[pl.ds(128, 72)], s2).wait()

        fire(0, rows_a, semA1, semA2)

        def row_body(i, carry):
            r0 = 2 * i
            r1 = r0 + 1
            r2 = jnp.minimum(r0 + 2, _RPW - 1)
            fire(r1, rows_b, semB1, semB2)
            wait(r0, rows_a, semA1, semA2)
            pltpu.sync_copy(rows_a, f_out.at[base + r0])
            fire(r2, rows_a, semA1, semA2)
            wait(r1, rows_b, semB1, semB2)
            pltpu.sync_copy(rows_b, f_out.at[base + r1])
            return carry

        lax.fori_loop(0, _RPW // 2, row_body, 0)
        wait(_RPW - 1, rows_a, semA1, semA2)

    return k(user_ids, item_ids, all_items, sample_items_padded)


_SC_CHUNK = 40
_JC = 8


def _tc_body(f_ref, emb_ref, uif_ref, samp_ref, g_ref, out1_ref, out2_ref, acc_ref):
    embb = emb_ref[...]
    uifb = uif_ref[...]
    sampb = samp_ref[...][:, :_S]
    gb = g_ref[...]

    rs_parts, ps_parts = [], []
    for s0 in range(0, _S, _SC_CHUNK):
        Fc = f_ref[:, s0:s0 + _SC_CHUNK, :]
        rs_parts.append(jnp.sum(Fc * embb[:, None, :], axis=-1))
        ps_parts.append(jnp.sum(Fc * uifb[:, None, :], axis=-1))
    rs = jnp.concatenate(rs_parts, axis=1)
    ps = jnp.concatenate(ps_parts, axis=1)

    logits = (ps + gb) / jnp.float32(1e-4)
    m = jnp.max(logits, axis=-1, keepdims=True)
    e = jnp.exp(logits - m)
    ysoft = e / jnp.sum(e, axis=-1, keepdims=True)
    ym = jnp.max(ysoft, axis=-1, keepdims=True)
    sidx = lax.broadcasted_iota(jnp.int32, (_R, _S), 1)
    amax = jnp.min(jnp.where(ysoft == ym, sidx, _S), axis=-1, keepdims=True)
    yhard = (sidx == amax).astype(jnp.float32)
    prob = yhard - ysoft + ysoft

    p_sel = jnp.sum(prob, axis=-1)

    bits = lax.bitcast_convert_type(rs, jnp.uint32)
    keys = bits ^ jnp.where(bits >= jnp.uint32(0x80000000),
                            jnp.uint32(0xFFFFFFFF), jnp.uint32(0x80000000))
    ones_s = jnp.ones((_S, 1), jnp.bfloat16)
    kidx3 = lax.broadcasted_iota(jnp.int32, (1, 1, _S), 2)
    kk = keys[:, None, :]
    cnt_parts = []
    for j0 in range(0, _S, _JC):
        kj = keys[:, j0:j0 + _JC][:, :, None]
        jj = j0 + lax.broadcasted_iota(jnp.int32, (1, _JC, 1), 1)
        ind = ((kk < kj) | ((kk == kj) & (kidx3 < jj))).astype(jnp.bfloat16)
        cc = lax.dot_general(ind.reshape(_R * _JC, _S), ones_s,
                             (((1,), (0,)), ((), ())),
                             preferred_element_type=jnp.float32)
        cnt_parts.append(cc.reshape(_R, _JC))
    cnt = jnp.concatenate(cnt_parts, axis=1)

    amax_f = amax.astype(jnp.float32)
    rank_am = jnp.sum(jnp.where(sidx == amax, cnt, 0.0), axis=-1)
    chosen = jnp.sum(jnp.where(cnt == amax_f, sampb.astype(jnp.float32), 0.0),
                     axis=-1)
    out1_ref[...] = (chosen * p_sel).astype(jnp.int32)[:, None]

    Fb = f_ref[...]
    out2_ref[...] = lax.dot_general(
        prob, Fb, (((1,), (1,)), ((0,), (0,))),
        precision=lax.Precision.HIGHEST)

    pos = rank_am * p_sel + 1.0
    sim = pos / jnp.float32(200.0)

    @pl.when(pl.program_id(0) == 0)
    def _init():
        acc_ref[...] = jnp.zeros_like(acc_ref)

    sa = jnp.sum(jnp.abs(sim - jnp.float32(0.5)))
    ss = jnp.sum(sim)
    r8 = lax.broadcasted_iota(jnp.int32, (8, 2), 0)
    c2 = lax.broadcasted_iota(jnp.int32, (8, 2), 1)
    upd = (jnp.where((r8 == 0) & (c2 == 0), sa, 0.0)
           + jnp.where((r8 == 0) & (c2 == 1), ss, 0.0))
    acc_ref[...] += upd


def _tc_compute(F, emb, uif, samp_pad, g):
    grid = (_B // _R,)
    return pl.pallas_call(
        _tc_body,
        grid=grid,
        in_specs=[
            pl.BlockSpec((_R, _S, _D), lambda i: (i, 0, 0)),
            pl.BlockSpec((_R, _D), lambda i: (i, 0)),
            pl.BlockSpec((_R, _D), lambda i: (i, 0)),
            pl.BlockSpec((_R, _SP), lambda i: (i, 0)),
            pl.BlockSpec((_R, _S), lambda i: (i, 0)),
        ],
        out_specs=[
            pl.BlockSpec((_R, 1), lambda i: (i, 0)),
            pl.BlockSpec((_R, _D), lambda i: (i, 0)),
            pl.BlockSpec((8, 2), lambda i: (0, 0)),
        ],
        out_shape=[
            jax.ShapeDtypeStruct((_B, 1), jnp.int32),
            jax.ShapeDtypeStruct((_B, _D), jnp.float32),
            jax.ShapeDtypeStruct((8, 2), jnp.float32),
        ],
        compiler_params=pltpu.CompilerParams(
            dimension_semantics=("arbitrary",),
        ),
    )(F, emb, uif, samp_pad, g)


def kernel(need_replace, union_feature, all_items, sample_items, W, b):
    user_ids = need_replace[:, 0]
    item_ids = need_replace[:, 1]
    uif = (union_feature @ W.T + b)
    u = jax.random.uniform(jax.random.key(42), (_B, _S), minval=1e-9, maxval=1.0)
    g = -jnp.log(-jnp.log(u))
    samp_src = jnp.pad(sample_items, ((0, 0), (0, _SP - _S)))
    F, samp_pad, emb = _sc_gather(user_ids, item_ids, all_items, samp_src)
    out1, out2, acc = _tc_compute(F, emb, uif, samp_pad, g)
    loss = acc[0, 0] / jnp.float32(_B)
    mean_sim = acc[0, 1] / jnp.float32(_B)
    return (out1.reshape(_B), out2, loss, mean_sim)


# speedup vs baseline: 4.2723x; 1.2408x over previous
import functools

import jax
import jax.numpy as jnp
from jax import lax
from jax.experimental import pallas as pl
from jax.experimental.pallas import tpu as pltpu
from jax.experimental.pallas import tpu_sc as plsc

_B = 4096
_S = 200
_SP = 256
_D = 128
_NC = 2
_NS = 16
_NW = _NC * _NS
_R = 32


def _sc_gather(user_ids, item_ids, all_items, sample_items_padded):
    mesh = plsc.VectorSubcoreMesh(core_axis_name="c", subcore_axis_name="s")
    n = user_ids.shape[0]
    _RPW = n // _NW

    @functools.partial(
        pl.kernel,
        out_type=(
            jax.ShapeDtypeStruct((n, _S, _D), jnp.float32),
            jax.ShapeDtypeStruct((n, _SP), jnp.int32),
            jax.ShapeDtypeStruct((n, _D), jnp.float32),
        ),
        mesh=mesh,
        scratch_types=[
            pltpu.VMEM((_RPW,), jnp.int32),
            pltpu.VMEM((_RPW,), jnp.int32),
            pltpu.VMEM((_RPW, _SP), jnp.int32),
            pltpu.VMEM((_RPW, _D), jnp.float32),
            pltpu.VMEM((_S, _D), jnp.float32),
            pltpu.VMEM((_S, _D), jnp.float32),
            pltpu.SemaphoreType.DMA,
            pltpu.SemaphoreType.DMA,
            pltpu.SemaphoreType.DMA,
            pltpu.SemaphoreType.DMA,
        ],
    )
    def k(uid_hbm, iid_hbm, items_hbm, samples_hbm, f_out, samp_out, emb_out,
          uid_v, iid_v, samp_v, emb_v, rows_a, rows_b, semA1, semA2, semB1, semB2):
        wid = lax.axis_index("s") * _NC + lax.axis_index("c")
        base = wid * _RPW
        pltpu.sync_copy(uid_hbm.at[pl.ds(base, _RPW)], uid_v)
        pltpu.sync_copy(iid_hbm.at[pl.ds(base, _RPW)], iid_v)
        pltpu.async_copy(samples_hbm.at[uid_v], samp_v, semA1).wait()
        pltpu.sync_copy(samp_v, samp_out.at[pl.ds(base, _RPW)])
        pltpu.async_copy(items_hbm.at[iid_v], emb_v, semA1).wait()
        pltpu.sync_copy(emb_v, emb_out.at[pl.ds(base, _RPW)])

        def fire(r, rv, s1, s2):
            pltpu.async_copy(items_hbm.at[samp_v.at[r, pl.ds(0, 128)]],
                             rv.at[pl.ds(0, 128)], s1)
            pltpu.async_copy(items_hbm.at[samp_v.at[r, pl.ds(128, 72)]],
                             rv.at[pl.ds(128, 72)], s2)

        def wait(r, rv, s1, s2):
            pltpu.make_async_copy(items_hbm.at[samp_v.at[r, pl.ds(0, 128)]],
                                  rv.at[pl.ds(0, 128)], s1).wait()
            pltpu.make_async_copy(items_hbm.at[samp_v.at[r, pl.ds(128, 72)]],
                                  rv.at[pl.ds(128, 72)], s2).wait()

        fire(0, rows_a, semA1, semA2)

        def row_body(i, carry):
            r0 = 2 * i
            r1 = r0 + 1
            r2 = jnp.minimum(r0 + 2, _RPW - 1)
            fire(r1, rows_b, semB1, semB2)
            wait(r0, rows_a, semA1, semA2)
            pltpu.sync_copy(rows_a, f_out.at[base + r0])
            fire(r2, rows_a, semA1, semA2)
            wait(r1, rows_b, semB1, semB2)
            pltpu.sync_copy(rows_b, f_out.at[base + r1])
            return carry

        lax.fori_loop(0, _RPW // 2, row_body, 0)
        wait(_RPW - 1, rows_a, semA1, semA2)

    return k(user_ids, item_ids, all_items, sample_items_padded)


_SC_CHUNK = 40
_JC = 8


def _tc_body(f_ref, emb_ref, uif_ref, samp_ref, g_ref, out1_ref, out2_ref, acc_ref):
    embb = emb_ref[...]
    uifb = uif_ref[...]
    sampb = samp_ref[...][:, :_S]
    gb = g_ref[...]

    rs_parts, ps_parts = [], []
    for s0 in range(0, _S, _SC_CHUNK):
        Fc = f_ref[:, s0:s0 + _SC_CHUNK, :]
        rs_parts.append(jnp.sum(Fc * embb[:, None, :], axis=-1))
        ps_parts.append(jnp.sum(Fc * uifb[:, None, :], axis=-1))
    rs = jnp.concatenate(rs_parts, axis=1)
    ps = jnp.concatenate(ps_parts, axis=1)

    logits = (ps + gb) / jnp.float32(1e-4)
    m = jnp.max(logits, axis=-1, keepdims=True)
    e = jnp.exp(logits - m)
    ysoft = e / jnp.sum(e, axis=-1, keepdims=True)
    ym = jnp.max(ysoft, axis=-1, keepdims=True)
    sidx = lax.broadcasted_iota(jnp.int32, (_R, _S), 1)
    amax = jnp.min(jnp.where(ysoft == ym, sidx, _S), axis=-1, keepdims=True)
    yhard = (sidx == amax).astype(jnp.float32)
    prob = yhard - ysoft + ysoft

    p_sel = jnp.sum(prob, axis=-1)

    bits = lax.bitcast_convert_type(rs, jnp.uint32)
    keys = bits ^ jnp.where(bits >= jnp.uint32(0x80000000),
                            jnp.uint32(0xFFFFFFFF), jnp.uint32(0x80000000))
    ones_s = jnp.ones((_S, 1), jnp.bfloat16)
    kidx3 = lax.broadcasted_iota(jnp.int32, (1, 1, _S), 2)
    kk = keys[:, None, :]
    cnt_parts = []
    for j0 in range(0, _S, _JC):
        kj = keys[:, j0:j0 + _JC][:, :, None]
        jj = j0 + lax.broadcasted_iota(jnp.int32, (1, _JC, 1), 1)
        ind = ((kk < kj) | ((kk == kj) & (kidx3 < jj))).astype(jnp.bfloat16)
        cc = lax.dot_general(ind.reshape(_R * _JC, _S), ones_s,
                             (((1,), (0,)), ((), ())),
                             preferred_element_type=jnp.float32)
        cnt_parts.append(cc.reshape(_R, _JC))
    cnt = jnp.concatenate(cnt_parts, axis=1)

    amax_f = amax.astype(jnp.float32)
    rank_am = jnp.sum(jnp.where(sidx == amax, cnt, 0.0), axis=-1)
    chosen = jnp.sum(jnp.where(cnt == amax_f, sampb.astype(jnp.float32), 0.0),
                     axis=-1)
    out1_ref[...] = (chosen * p_sel).astype(jnp.int32)[:, None]

    Fb = f_ref[...]
    out2_ref[...] = lax.dot_general(
        prob, Fb, (((1,), (1,)), ((0,), (0,))),
        precision=lax.Precision.HIGHEST)

    pos = rank_am * p_sel + 1.0
    sim = pos / jnp.float32(200.0)

    @pl.when(pl.program_id(0) == 0)
    def _init():
        acc_ref[...] = jnp.zeros_like(acc_ref)

    sa = jnp.sum(jnp.abs(sim - jnp.float32(0.5)))
    ss = jnp.sum(sim)
    r8 = lax.broadcasted_iota(jnp.int32, (8, 2), 0)
    c2 = lax.broadcasted_iota(jnp.int32, (8, 2), 1)
    upd = (jnp.where((r8 == 0) & (c2 == 0), sa, 0.0)
           + jnp.where((r8 == 0) & (c2 == 1), ss, 0.0))
    acc_ref[...] += upd


def _tc_compute(F, emb, uif, samp_pad, g):
    n = F.shape[0]
    grid = (n // _R,)
    return pl.pallas_call(
        _tc_body,
        grid=grid,
        in_specs=[
            pl.BlockSpec((_R, _S, _D), lambda i: (i, 0, 0)),
            pl.BlockSpec((_R, _D), lambda i: (i, 0)),
            pl.BlockSpec((_R, _D), lambda i: (i, 0)),
            pl.BlockSpec((_R, _SP), lambda i: (i, 0)),
            pl.BlockSpec((_R, _S), lambda i: (i, 0)),
        ],
        out_specs=[
            pl.BlockSpec((_R, 1), lambda i: (i, 0)),
            pl.BlockSpec((_R, _D), lambda i: (i, 0)),
            pl.BlockSpec((8, 2), lambda i: (0, 0)),
        ],
        out_shape=[
            jax.ShapeDtypeStruct((n, 1), jnp.int32),
            jax.ShapeDtypeStruct((n, _D), jnp.float32),
            jax.ShapeDtypeStruct((8, 2), jnp.float32),
        ],
        compiler_params=pltpu.CompilerParams(
            dimension_semantics=("arbitrary",),
        ),
    )(F, emb, uif, samp_pad, g)


def kernel(need_replace, union_feature, all_items, sample_items, W, b):
    user_ids = need_replace[:, 0]
    item_ids = need_replace[:, 1]
    uif = (union_feature @ W.T + b)
    u = jax.random.uniform(jax.random.key(42), (_B, _S), minval=1e-9, maxval=1.0)
    g = -jnp.log(-jnp.log(u))
    samp_src = jnp.pad(sample_items, ((0, 0), (0, _SP - _S)))
    h = _B // 2
    gathered = []
    for i in range(2):
        sl = slice(i * h, (i + 1) * h)
        gathered.append(_sc_gather(user_ids[sl], item_ids[sl], all_items, samp_src))
    outs = []
    for i in range(2):
        sl = slice(i * h, (i + 1) * h)
        F, samp_pad, emb = gathered[i]
        outs.append(_tc_compute(F, emb, uif[sl], samp_pad, g[sl]))
    out1 = jnp.concatenate([o[0] for o in outs], axis=0)
    out2 = jnp.concatenate([o[1] for o in outs], axis=0)
    acc = outs[0][2] + outs[1][2]
    loss = acc[0, 0] / jnp.float32(_B)
    mean_sim = acc[0, 1] / jnp.float32(_B)
    return (out1.reshape(_B), out2, loss, mean_sim)


# speedup vs baseline: 4.9691x; 1.1631x over previous
import functools

import jax
import jax.numpy as jnp
from jax import lax
from jax.experimental import pallas as pl
from jax.experimental.pallas import tpu as pltpu
from jax.experimental.pallas import tpu_sc as plsc

_B = 4096
_S = 200
_SP = 256
_D = 128
_NC = 2
_NS = 16
_NW = _NC * _NS
_R = 32


def _sc_gather(user_ids, item_ids, all_items, sample_items_padded):
    mesh = plsc.VectorSubcoreMesh(core_axis_name="c", subcore_axis_name="s")
    n = user_ids.shape[0]
    _RPW = n // _NW

    @functools.partial(
        pl.kernel,
        out_type=(
            jax.ShapeDtypeStruct((n, _S, _D), jnp.float32),
            jax.ShapeDtypeStruct((n, _SP), jnp.int32),
            jax.ShapeDtypeStruct((n, _D), jnp.float32),
        ),
        mesh=mesh,
        scratch_types=[
            pltpu.VMEM((_RPW,), jnp.int32),
            pltpu.VMEM((_RPW,), jnp.int32),
            pltpu.VMEM((_RPW, _SP), jnp.int32),
            pltpu.VMEM((_RPW, _D), jnp.float32),
            pltpu.VMEM((_S, _D), jnp.float32),
            pltpu.VMEM((_S, _D), jnp.float32),
            pltpu.SemaphoreType.DMA,
            pltpu.SemaphoreType.DMA,
            pltpu.SemaphoreType.DMA,
            pltpu.SemaphoreType.DMA,
        ],
    )
    def k(uid_hbm, iid_hbm, items_hbm, samples_hbm, f_out, samp_out, emb_out,
          uid_v, iid_v, samp_v, emb_v, rows_a, rows_b, semA1, semA2, semB1, semB2):
        wid = lax.axis_index("s") * _NC + lax.axis_index("c")
        base = wid * _RPW
        pltpu.sync_copy(uid_hbm.at[pl.ds(base, _RPW)], uid_v)
        pltpu.sync_copy(iid_hbm.at[pl.ds(base, _RPW)], iid_v)
        pltpu.async_copy(samples_hbm.at[uid_v], samp_v, semA1).wait()
        pltpu.sync_copy(samp_v, samp_out.at[pl.ds(base, _RPW)])
        pltpu.async_copy(items_hbm.at[iid_v], emb_v, semA1).wait()
        pltpu.sync_copy(emb_v, emb_out.at[pl.ds(base, _RPW)])

        def fire(r, rv, s1, s2):
            pltpu.async_copy(items_hbm.at[samp_v.at[r, pl.ds(0, 128)]],
                             rv.at[pl.ds(0, 128)], s1)
            pltpu.async_copy(items_hbm.at[samp_v.at[r, pl.ds(128, 72)]],
                             rv.at[pl.ds(128, 72)], s2)

        def wait(r, rv, s1, s2):
            pltpu.make_async_copy(items_hbm.at[samp_v.at[r, pl.ds(0, 128)]],
                                  rv.at[pl.ds(0, 128)], s1).wait()
            pltpu.make_async_copy(items_hbm.at[samp_v.at[r, pl.ds(128, 72)]],
                                  rv.at[pl.ds(128, 72)], s2).wait()

        fire(0, rows_a, semA1, semA2)

        def row_body(i, carry):
            r0 = 2 * i
            r1 = r0 + 1
            r2 = jnp.minimum(r0 + 2, _RPW - 1)
            fire(r1, rows_b, semB1, semB2)
            wait(r0, rows_a, semA1, semA2)
            pltpu.sync_copy(rows_a, f_out.at[base + r0])
            fire(r2, rows_a, semA1, semA2)
            wait(r1, rows_b, semB1, semB2)
            pltpu.sync_copy(rows_b, f_out.at[base + r1])
            return carry

        lax.fori_loop(0, _RPW // 2, row_body, 0)
        wait(_RPW - 1, rows_a, semA1, semA2)

    return k(user_ids, item_ids, all_items, sample_items_padded)


_SC_CHUNK = 40
_JC = 8


def _tc_body(f_ref, emb_ref, uif_ref, samp_ref, g_ref, out1_ref, out2_ref, acc_ref):
    embb = emb_ref[...]
    uifb = uif_ref[...]
    sampb = samp_ref[...][:, :_S]
    gb = g_ref[...]

    rs_parts, ps_parts = [], []
    for s0 in range(0, _S, _SC_CHUNK):
        Fc = f_ref[:, s0:s0 + _SC_CHUNK, :]
        rs_parts.append(jnp.sum(Fc * embb[:, None, :], axis=-1))
        ps_parts.append(jnp.sum(Fc * uifb[:, None, :], axis=-1))
    rs = jnp.concatenate(rs_parts, axis=1)
    ps = jnp.concatenate(ps_parts, axis=1)

    logits = (ps + gb) / jnp.float32(1e-4)
    m = jnp.max(logits, axis=-1, keepdims=True)
    e = jnp.exp(logits - m)
    ysoft = e / jnp.sum(e, axis=-1, keepdims=True)
    ym = jnp.max(ysoft, axis=-1, keepdims=True)
    sidx = lax.broadcasted_iota(jnp.int32, (_R, _S), 1)
    amax = jnp.min(jnp.where(ysoft == ym, sidx, _S), axis=-1, keepdims=True)
    yhard = (sidx == amax).astype(jnp.float32)
    prob = yhard - ysoft + ysoft

    p_sel = jnp.sum(prob, axis=-1)

    bits = lax.bitcast_convert_type(rs, jnp.uint32)
    keys = bits ^ jnp.where(bits >= jnp.uint32(0x80000000),
                            jnp.uint32(0xFFFFFFFF), jnp.uint32(0x80000000))
    ones_s = jnp.ones((_S, 1), jnp.bfloat16)
    kidx3 = lax.broadcasted_iota(jnp.int32, (1, 1, _S), 2)
    kk = keys[:, None, :]
    cnt_parts = []
    for j0 in range(0, _S, _JC):
        kj = keys[:, j0:j0 + _JC][:, :, None]
        jj = j0 + lax.broadcasted_iota(jnp.int32, (1, _JC, 1), 1)
        ind = ((kk < kj) | ((kk == kj) & (kidx3 < jj))).astype(jnp.bfloat16)
        cc = lax.dot_general(ind.reshape(_R * _JC, _S), ones_s,
                             (((1,), (0,)), ((), ())),
                             preferred_element_type=jnp.float32)
        cnt_parts.append(cc.reshape(_R, _JC))
    cnt = jnp.concatenate(cnt_parts, axis=1)

    amax_f = amax.astype(jnp.float32)
    rank_am = jnp.sum(jnp.where(sidx == amax, cnt, 0.0), axis=-1)
    chosen = jnp.sum(jnp.where(cnt == amax_f, sampb.astype(jnp.float32), 0.0),
                     axis=-1)
    out1_ref[...] = (chosen * p_sel).astype(jnp.int32)[:, None]

    Fb = f_ref[...]
    out2_ref[...] = lax.dot_general(prob, Fb, (((1,), (1,)), ((0,), (0,))))

    pos = rank_am * p_sel + 1.0
    sim = pos / jnp.float32(200.0)

    @pl.when(pl.program_id(0) == 0)
    def _init():
        acc_ref[...] = jnp.zeros_like(acc_ref)

    sa = jnp.sum(jnp.abs(sim - jnp.float32(0.5)))
    ss = jnp.sum(sim)
    r8 = lax.broadcasted_iota(jnp.int32, (8, 2), 0)
    c2 = lax.broadcasted_iota(jnp.int32, (8, 2), 1)
    upd = (jnp.where((r8 == 0) & (c2 == 0), sa, 0.0)
           + jnp.where((r8 == 0) & (c2 == 1), ss, 0.0))
    acc_ref[...] += upd


def _tc_compute(F, emb, uif, samp_pad, g):
    n = F.shape[0]
    grid = (n // _R,)
    return pl.pallas_call(
        _tc_body,
        grid=grid,
        in_specs=[
            pl.BlockSpec((_R, _S, _D), lambda i: (i, 0, 0)),
            pl.BlockSpec((_R, _D), lambda i: (i, 0)),
            pl.BlockSpec((_R, _D), lambda i: (i, 0)),
            pl.BlockSpec((_R, _SP), lambda i: (i, 0)),
            pl.BlockSpec((_R, _S), lambda i: (i, 0)),
        ],
        out_specs=[
            pl.BlockSpec((_R, 1), lambda i: (i, 0)),
            pl.BlockSpec((_R, _D), lambda i: (i, 0)),
            pl.BlockSpec((8, 2), lambda i: (0, 0)),
        ],
        out_shape=[
            jax.ShapeDtypeStruct((n, 1), jnp.int32),
            jax.ShapeDtypeStruct((n, _D), jnp.float32),
            jax.ShapeDtypeStruct((8, 2), jnp.float32),
        ],
        compiler_params=pltpu.CompilerParams(
            dimension_semantics=("arbitrary",),
        ),
    )(F, emb, uif, samp_pad, g)


def kernel(need_replace, union_feature, all_items, sample_items, W, b):
    user_ids = need_replace[:, 0]
    item_ids = need_replace[:, 1]
    uif = (union_feature @ W.T + b)
    u = jax.random.uniform(jax.random.key(42), (_B, _S), minval=1e-9, maxval=1.0)
    g = -jnp.log(-jnp.log(u))
    samp_src = jnp.pad(sample_items, ((0, 0), (0, _SP - _S)))
    h = _B // 2
    gathered = []
    for i in range(2):
        sl = slice(i * h, (i + 1) * h)
        gathered.append(_sc_gather(user_ids[sl], item_ids[sl], all_items, samp_src))
    outs = []
    for i in range(2):
        sl = slice(i * h, (i + 1) * h)
        F, samp_pad, emb = gathered[i]
        outs.append(_tc_compute(F, emb, uif[sl], samp_pad, g[sl]))
    out1 = jnp.concatenate([o[0] for o in outs], axis=0)
    out2 = jnp.concatenate([o[1] for o in outs], axis=0)
    acc = outs[0][2] + outs[1][2]
    loss = acc[0, 0] / jnp.float32(_B)
    mean_sim = acc[0, 1] / jnp.float32(_B)
    return (out1.reshape(_B), out2, loss, mean_sim)


# speedup vs baseline: 5.3308x; 1.0728x over previous
import functools

import jax
import jax.numpy as jnp
from jax import lax
from jax.experimental import pallas as pl
from jax.experimental.pallas import tpu as pltpu
from jax.experimental.pallas import tpu_sc as plsc

_B = 4096
_S = 200
_SP = 256
_D = 128
_NC = 2
_NS = 16
_NW = _NC * _NS
_R = 32


def _sc_gather(user_ids, item_ids, all_items, sample_items_padded):
    mesh = plsc.VectorSubcoreMesh(core_axis_name="c", subcore_axis_name="s")
    n = user_ids.shape[0]
    _RPW = n // _NW

    @functools.partial(
        pl.kernel,
        out_type=(
            jax.ShapeDtypeStruct((n, _S, _D), jnp.float32),
            jax.ShapeDtypeStruct((n, _SP), jnp.int32),
            jax.ShapeDtypeStruct((n, _D), jnp.float32),
        ),
        mesh=mesh,
        scratch_types=[
            pltpu.VMEM((_RPW,), jnp.int32),
            pltpu.VMEM((_RPW,), jnp.int32),
            pltpu.VMEM((_RPW, _SP), jnp.int32),
            pltpu.VMEM((_RPW, _D), jnp.float32),
            pltpu.VMEM((_S, _D), jnp.float32),
            pltpu.VMEM((_S, _D), jnp.float32),
            pltpu.SemaphoreType.DMA,
            pltpu.SemaphoreType.DMA,
            pltpu.SemaphoreType.DMA,
            pltpu.SemaphoreType.DMA,
        ],
    )
    def k(uid_hbm, iid_hbm, items_hbm, samples_hbm, f_out, samp_out, emb_out,
          uid_v, iid_v, samp_v, emb_v, rows_a, rows_b, semA1, semA2, semB1, semB2):
        wid = lax.axis_index("s") * _NC + lax.axis_index("c")
        base = wid * _RPW
        pltpu.sync_copy(uid_hbm.at[pl.ds(base, _RPW)], uid_v)
        pltpu.sync_copy(iid_hbm.at[pl.ds(base, _RPW)], iid_v)
        pltpu.async_copy(samples_hbm.at[uid_v], samp_v, semA1).wait()
        pltpu.sync_copy(samp_v, samp_out.at[pl.ds(base, _RPW)])
        pltpu.async_copy(items_hbm.at[iid_v], emb_v, semA1).wait()
        pltpu.sync_copy(emb_v, emb_out.at[pl.ds(base, _RPW)])

        def fire(r, rv, s1, s2):
            pltpu.async_copy(items_hbm.at[samp_v.at[r, pl.ds(0, 128)]],
                             rv.at[pl.ds(0, 128)], s1)
            pltpu.async_copy(items_hbm.at[samp_v.at[r, pl.ds(128, 72)]],
                             rv.at[pl.ds(128, 72)], s2)

        def wait(r, rv, s1, s2):
            pltpu.make_async_copy(items_hbm.at[samp_v.at[r, pl.ds(0, 128)]],
                                  rv.at[pl.ds(0, 128)], s1).wait()
            pltpu.make_async_copy(items_hbm.at[samp_v.at[r, pl.ds(128, 72)]],
                                  rv.at[pl.ds(128, 72)], s2).wait()

        fire(0, rows_a, semA1, semA2)

        def row_body(i, carry):
            r0 = 2 * i
            r1 = r0 + 1
            r2 = jnp.minimum(r0 + 2, _RPW - 1)
            fire(r1, rows_b, semB1, semB2)
            wait(r0, rows_a, semA1, semA2)
            pltpu.sync_copy(rows_a, f_out.at[base + r0])
            fire(r2, rows_a, semA1, semA2)
            wait(r1, rows_b, semB1, semB2)
            pltpu.sync_copy(rows_b, f_out.at[base + r1])
            return carry

        lax.fori_loop(0, _RPW // 2, row_body, 0)
        wait(_RPW - 1, rows_a, semA1, semA2)

    return k(user_ids, item_ids, all_items, sample_items_padded)


_SC_CHUNK = 40
_JC = 8


def _tc_body(f_ref, emb_ref, uif_ref, samp_ref, g_ref, out1_ref, out2_ref, acc_ref):
    embb = emb_ref[...]
    uifb = uif_ref[...]
    sampb = samp_ref[...][:, :_S]
    gb = g_ref[...]

    rs_parts, ps_parts = [], []
    for s0 in range(0, _S, _SC_CHUNK):
        Fc = f_ref[:, s0:s0 + _SC_CHUNK, :]
        rs_parts.append(jnp.sum(Fc * embb[:, None, :], axis=-1))
        ps_parts.append(jnp.sum(Fc * uifb[:, None, :], axis=-1))
    rs = jnp.concatenate(rs_parts, axis=1)
    ps = jnp.concatenate(ps_parts, axis=1)

    logits = (ps + gb) / jnp.float32(1e-4)
    m = jnp.max(logits, axis=-1, keepdims=True)
    e = jnp.exp(logits - m)
    ysoft = e / jnp.sum(e, axis=-1, keepdims=True)
    ym = jnp.max(ysoft, axis=-1, keepdims=True)
    sidx = lax.broadcasted_iota(jnp.int32, (_R, _S), 1)
    amax = jnp.min(jnp.where(ysoft == ym, sidx, _S), axis=-1, keepdims=True)
    yhard = (sidx == amax).astype(jnp.float32)
    prob = yhard - ysoft + ysoft

    p_sel = jnp.sum(prob, axis=-1)

    bits = lax.bitcast_convert_type(rs, jnp.uint32)
    keys = bits ^ jnp.where(bits >= jnp.uint32(0x80000000),
                            jnp.uint32(0xFFFFFFFF), jnp.uint32(0x80000000))
    ones_s = jnp.ones((_S, 1), jnp.bfloat16)
    kidx3 = lax.broadcasted_iota(jnp.int32, (1, 1, _S), 2)
    kk = keys[:, None, :]
    cnt_parts = []
    for j0 in range(0, _S, _JC):
        kj = keys[:, j0:j0 + _JC][:, :, None]
        jj = j0 + lax.broadcasted_iota(jnp.int32, (1, _JC, 1), 1)
        ind = ((kk < kj) | ((kk == kj) & (kidx3 < jj))).astype(jnp.bfloat16)
        cc = lax.dot_general(ind.reshape(_R * _JC, _S), ones_s,
                             (((1,), (0,)), ((), ())),
                             preferred_element_type=jnp.float32)
        cnt_parts.append(cc.reshape(_R, _JC))
    cnt = jnp.concatenate(cnt_parts, axis=1)

    amax_f = amax.astype(jnp.float32)
    rank_am = jnp.sum(jnp.where(sidx == amax, cnt, 0.0), axis=-1)
    chosen = jnp.sum(jnp.where(cnt == amax_f, sampb.astype(jnp.float32), 0.0),
                     axis=-1)
    out1_ref[...] = (chosen * p_sel).astype(jnp.int32)[:, None]

    Fb = f_ref[...]
    out2_ref[...] = lax.dot_general(prob, Fb, (((1,), (1,)), ((0,), (0,))))

    pos = rank_am * p_sel + 1.0
    sim = pos / jnp.float32(200.0)

    @pl.when(pl.program_id(0) == 0)
    def _init():
        acc_ref[...] = jnp.zeros_like(acc_ref)

    sa = jnp.sum(jnp.abs(sim - jnp.float32(0.5)))
    ss = jnp.sum(sim)
    r8 = lax.broadcasted_iota(jnp.int32, (8, 2), 0)
    c2 = lax.broadcasted_iota(jnp.int32, (8, 2), 1)
    upd = (jnp.where((r8 == 0) & (c2 == 0), sa, 0.0)
           + jnp.where((r8 == 0) & (c2 == 1), ss, 0.0))
    acc_ref[...] += upd


def _tc_compute(F, emb, uif, samp_pad, g):
    n = F.shape[0]
    grid = (n // _R,)
    return pl.pallas_call(
        _tc_body,
        grid=grid,
        in_specs=[
            pl.BlockSpec((_R, _S, _D), lambda i: (i, 0, 0)),
            pl.BlockSpec((_R, _D), lambda i: (i, 0)),
            pl.BlockSpec((_R, _D), lambda i: (i, 0)),
            pl.BlockSpec((_R, _SP), lambda i: (i, 0)),
            pl.BlockSpec((_R, _S), lambda i: (i, 0)),
        ],
        out_specs=[
            pl.BlockSpec((_R, 1), lambda i: (i, 0)),
            pl.BlockSpec((_R, _D), lambda i: (i, 0)),
            pl.BlockSpec((8, 2), lambda i: (0, 0)),
        ],
        out_shape=[
            jax.ShapeDtypeStruct((n, 1), jnp.int32),
            jax.ShapeDtypeStruct((n, _D), jnp.float32),
            jax.ShapeDtypeStruct((8, 2), jnp.float32),
        ],
        compiler_params=pltpu.CompilerParams(
            dimension_semantics=("arbitrary",),
        ),
    )(F, emb, uif, samp_pad, g)


def kernel(need_replace, union_feature, all_items, sample_items, W, b):
    user_ids = need_replace[:, 0]
    item_ids = need_replace[:, 1]
    uif = (union_feature @ W.T + b)
    u = jax.random.uniform(jax.random.key(42), (_B, _S), minval=1e-9, maxval=1.0)
    g = -jnp.log(-jnp.log(u))
    samp_src = jnp.pad(sample_items, ((0, 0), (0, _SP - _S)))
    nsplit = 4
    h = _B // nsplit
    gathered = []
    for i in range(nsplit):
        sl = slice(i * h, (i + 1) * h)
        gathered.append(_sc_gather(user_ids[sl], item_ids[sl], all_items, samp_src))
    outs = []
    for i in range(nsplit):
        sl = slice(i * h, (i + 1) * h)
        F, samp_pad, emb = gathered[i]
        outs.append(_tc_compute(F, emb, uif[sl], samp_pad, g[sl]))
    out1 = jnp.concatenate([o[0] for o in outs], axis=0)
    out2 = jnp.concatenate([o[1] for o in outs], axis=0)
    acc = outs[0][2]
    for o in outs[1:]:
        acc = acc + o[2]
    loss = acc[0, 0] / jnp.float32(_B)
    mean_sim = acc[0, 1] / jnp.float32(_B)
    return (out1.reshape(_B), out2, loss, mean_sim)


# speedup vs baseline: 5.3625x; 1.0060x over previous
import functools

import jax
import jax.numpy as jnp
from jax import lax
from jax.experimental import pallas as pl
from jax.experimental.pallas import tpu as pltpu
from jax.experimental.pallas import tpu_sc as plsc

_B = 4096
_S = 200
_SP = 256
_D = 128
_NC = 2
_NS = 16
_NW = _NC * _NS
_R = 32


def _sc_gather(user_ids, item_ids, all_items, sample_items_padded):
    mesh = plsc.VectorSubcoreMesh(core_axis_name="c", subcore_axis_name="s")
    n = user_ids.shape[0]
    _RPW = n // _NW

    @functools.partial(
        pl.kernel,
        out_type=(
            jax.ShapeDtypeStruct((n, _S, _D), jnp.float32),
            jax.ShapeDtypeStruct((n, _SP), jnp.int32),
            jax.ShapeDtypeStruct((n, _D), jnp.float32),
        ),
        mesh=mesh,
        scratch_types=[
            pltpu.VMEM((_RPW,), jnp.int32),
            pltpu.VMEM((_RPW,), jnp.int32),
            pltpu.VMEM((_RPW, _SP), jnp.int32),
            pltpu.VMEM((_RPW, _D), jnp.float32),
            pltpu.VMEM((_S, _D), jnp.float32),
            pltpu.VMEM((_S, _D), jnp.float32),
            pltpu.SemaphoreType.DMA,
            pltpu.SemaphoreType.DMA,
            pltpu.SemaphoreType.DMA,
            pltpu.SemaphoreType.DMA,
        ],
    )
    def k(uid_hbm, iid_hbm, items_hbm, samples_hbm, f_out, samp_out, emb_out,
          uid_v, iid_v, samp_v, emb_v, rows_a, rows_b, semA1, semA2, semB1, semB2):
        wid = lax.axis_index("s") * _NC + lax.axis_index("c")
        base = wid * _RPW
        pltpu.sync_copy(uid_hbm.at[pl.ds(base, _RPW)], uid_v)
        pltpu.sync_copy(iid_hbm.at[pl.ds(base, _RPW)], iid_v)
        pltpu.async_copy(samples_hbm.at[uid_v], samp_v, semA1).wait()
        pltpu.sync_copy(samp_v, samp_out.at[pl.ds(base, _RPW)])
        pltpu.async_copy(items_hbm.at[iid_v], emb_v, semA1).wait()
        pltpu.sync_copy(emb_v, emb_out.at[pl.ds(base, _RPW)])

        def fire(r, rv, s1, s2):
            pltpu.async_copy(items_hbm.at[samp_v.at[r, pl.ds(0, 128)]],
                             rv.at[pl.ds(0, 128)], s1)
            pltpu.async_copy(items_hbm.at[samp_v.at[r, pl.ds(128, 72)]],
                             rv.at[pl.ds(128, 72)], s2)

        def wait(r, rv, s1, s2):
            pltpu.make_async_copy(items_hbm.at[samp_v.at[r, pl.ds(0, 128)]],
                                  rv.at[pl.ds(0, 128)], s1).wait()
            pltpu.make_async_copy(items_hbm.at[samp_v.at[r, pl.ds(128, 72)]],
                                  rv.at[pl.ds(128, 72)], s2).wait()

        fire(0, rows_a, semA1, semA2)

        def row_body(i, carry):
            r0 = 2 * i
            r1 = r0 + 1
            r2 = jnp.minimum(r0 + 2, _RPW - 1)
            fire(r1, rows_b, semB1, semB2)
            wait(r0, rows_a, semA1, semA2)
            pltpu.sync_copy(rows_a, f_out.at[base + r0])
            fire(r2, rows_a, semA1, semA2)
            wait(r1, rows_b, semB1, semB2)
            pltpu.sync_copy(rows_b, f_out.at[base + r1])
            return carry

        lax.fori_loop(0, _RPW // 2, row_body, 0)
        wait(_RPW - 1, rows_a, semA1, semA2)

    return k(user_ids, item_ids, all_items, sample_items_padded)


_SC_CHUNK = 40
_JC = 8


def _tc_body(f_ref, emb_ref, uif_ref, samp_ref, g_ref, out1_ref, out2_ref, acc_ref):
    embb = emb_ref[...]
    uifb = uif_ref[...]
    sampb = samp_ref[...][:, :_S]
    gb = g_ref[...]

    rs_parts, ps_parts = [], []
    for s0 in range(0, _S, _SC_CHUNK):
        Fc = f_ref[:, s0:s0 + _SC_CHUNK, :]
        rs_parts.append(jnp.sum(Fc * embb[:, None, :], axis=-1))
        ps_parts.append(jnp.sum(Fc * uifb[:, None, :], axis=-1))
    rs = jnp.concatenate(rs_parts, axis=1)
    ps = jnp.concatenate(ps_parts, axis=1)

    logits = (ps + gb) / jnp.float32(1e-4)
    m = jnp.max(logits, axis=-1, keepdims=True)
    e = jnp.exp(logits - m)
    ysoft = e / jnp.sum(e, axis=-1, keepdims=True)
    ym = jnp.max(ysoft, axis=-1, keepdims=True)
    sidx = lax.broadcasted_iota(jnp.int32, (_R, _S), 1)
    amax = jnp.min(jnp.where(ysoft == ym, sidx, _S), axis=-1, keepdims=True)
    yhard = (sidx == amax).astype(jnp.float32)
    prob = yhard - ysoft + ysoft

    p_sel = jnp.sum(prob, axis=-1)

    bits = lax.bitcast_convert_type(rs, jnp.uint32)
    keys = bits ^ jnp.where(bits >= jnp.uint32(0x80000000),
                            jnp.uint32(0xFFFFFFFF), jnp.uint32(0x80000000))
    ones_s = jnp.ones((_S, 1), jnp.bfloat16)
    kidx3 = lax.broadcasted_iota(jnp.int32, (1, 1, _S), 2)
    kk = keys[:, None, :]
    cnt_parts = []
    for j0 in range(0, _S, _JC):
        kj = keys[:, j0:j0 + _JC][:, :, None]
        jj = j0 + lax.broadcasted_iota(jnp.int32, (1, _JC, 1), 1)
        ind = ((kk < kj) | ((kk == kj) & (kidx3 < jj))).astype(jnp.bfloat16)
        cc = lax.dot_general(ind.reshape(_R * _JC, _S), ones_s,
                             (((1,), (0,)), ((), ())),
                             preferred_element_type=jnp.float32)
        cnt_parts.append(cc.reshape(_R, _JC))
    cnt = jnp.concatenate(cnt_parts, axis=1)

    amax_f = amax.astype(jnp.float32)
    rank_am = jnp.sum(jnp.where(sidx == amax, cnt, 0.0), axis=-1)
    chosen = jnp.sum(jnp.where(cnt == amax_f, sampb.astype(jnp.float32), 0.0),
                     axis=-1)
    out1_ref[...] = (chosen * p_sel).astype(jnp.int32)[:, None]

    Fb = f_ref[...]
    out2_ref[...] = lax.dot_general(prob, Fb, (((1,), (1,)), ((0,), (0,))))

    pos = rank_am * p_sel + 1.0
    sim = pos / jnp.float32(200.0)

    @pl.when(pl.program_id(0) == 0)
    def _init():
        acc_ref[...] = jnp.zeros_like(acc_ref)

    sa = jnp.sum(jnp.abs(sim - jnp.float32(0.5)))
    ss = jnp.sum(sim)
    r8 = lax.broadcasted_iota(jnp.int32, (8, 2), 0)
    c2 = lax.broadcasted_iota(jnp.int32, (8, 2), 1)
    upd = (jnp.where((r8 == 0) & (c2 == 0), sa, 0.0)
           + jnp.where((r8 == 0) & (c2 == 1), ss, 0.0))
    acc_ref[...] += upd


def _tc_compute(F, emb, uif, samp_pad, g):
    n = F.shape[0]
    grid = (n // _R,)
    return pl.pallas_call(
        _tc_body,
        grid=grid,
        in_specs=[
            pl.BlockSpec((_R, _S, _D), lambda i: (i, 0, 0)),
            pl.BlockSpec((_R, _D), lambda i: (i, 0)),
            pl.BlockSpec((_R, _D), lambda i: (i, 0)),
            pl.BlockSpec((_R, _SP), lambda i: (i, 0)),
            pl.BlockSpec((_R, _S), lambda i: (i, 0)),
        ],
        out_specs=[
            pl.BlockSpec((_R, 1), lambda i: (i, 0)),
            pl.BlockSpec((_R, _D), lambda i: (i, 0)),
            pl.BlockSpec((8, 2), lambda i: (0, 0)),
        ],
        out_shape=[
            jax.ShapeDtypeStruct((n, 1), jnp.int32),
            jax.ShapeDtypeStruct((n, _D), jnp.float32),
            jax.ShapeDtypeStruct((8, 2), jnp.float32),
        ],
        compiler_params=pltpu.CompilerParams(
            dimension_semantics=("arbitrary",),
        ),
    )(F, emb, uif, samp_pad, g)


def kernel(need_replace, union_feature, all_items, sample_items, W, b):
    user_ids = need_replace[:, 0]
    item_ids = need_replace[:, 1]
    uif = (union_feature @ W.T + b)
    u = jax.random.uniform(jax.random.key(42), (_B, _S), minval=1e-9, maxval=1.0)
    g = -jnp.log(-jnp.log(u))
    samp_src = jnp.pad(sample_items, ((0, 0), (0, _SP - _S)))
    nsplit = 8
    h = _B // nsplit
    gathered = []
    for i in range(nsplit):
        sl = slice(i * h, (i + 1) * h)
        gathered.append(_sc_gather(user_ids[sl], item_ids[sl], all_items, samp_src))
    outs = []
    for i in range(nsplit):
        sl = slice(i * h, (i + 1) * h)
        F, samp_pad, emb = gathered[i]
        outs.append(_tc_compute(F, emb, uif[sl], samp_pad, g[sl]))
    out1 = jnp.concatenate([o[0] for o in outs], axis=0)
    out2 = jnp.concatenate([o[1] for o in outs], axis=0)
    acc = outs[0][2]
    for o in outs[1:]:
        acc = acc + o[2]
    loss = acc[0, 0] / jnp.float32(_B)
    mean_sim = acc[0, 1] / jnp.float32(_B)
    return (out1.reshape(_B), out2, loss, mean_sim)
